# baseline (speedup 1.0000x reference)
import functools
import jax
import jax.numpy as jnp
from jax import lax
from jax.experimental import pallas as pl
from jax.experimental.pallas import tpu as pltpu
from jax.experimental.pallas import tpu_sc as plsc

_EPS = 1e-5
_NB1 = 16384
_NB2 = 4096
_NEG = -1.0e9
_KAUG = 72


def _stage1_body(x_ref, ring_ref, aaug_ref, out_ref, acc_ref, *, nb, nj):
    j = pl.program_id(1)

    xb = x_ref[0].astype(jnp.bfloat16)
    r = ring_ref[0, 0]
    madd0 = jnp.where(r == 0, 0.0, _NEG)
    madd1 = jnp.where(r == 0, _NEG, 0.0)
    srow = jax.lax.broadcasted_iota(jnp.int32, (_KAUG - 64, nb), 0)
    mrows = jnp.where(srow == 0, madd0[None, :],
                      jnp.where(srow == 1, madd1[None, :], 0.0))
    xaug = jnp.concatenate([xb, mrows.astype(jnp.bfloat16)], axis=0)
    h = jnp.dot(aaug_ref[...], xaug, preferred_element_type=jnp.float32)
    parts = [h[:, k * 128:(k + 1) * 128] for k in range(nb // 128)]
    while len(parts) > 1:
        nxt = [jnp.maximum(parts[i], parts[i + 1]) for i in range(0, len(parts) - 1, 2)]
        if len(parts) % 2:
            nxt.append(parts[-1])
        parts = nxt
    p = parts[0]

    @pl.when(j == 0)
    def _():
        acc_ref[...] = p

    @pl.when(j > 0)
    def _():
        acc_ref[...] = jnp.maximum(acc_ref[...], p)

    @pl.when(j == nj - 1)
    def _():
        out_ref[0, 0] = jnp.max(acc_ref[...], axis=1)


_SC_NC = 2
_SC_NS = 16
_SC_L = 16
_SC_CH = 512


def _sc_stage2_body(mtab_hbm, ring_hbm, out_hbm, rv, obuf0, obuf1, tab,
                    sem0, sem1, *, Bx, Dout, Nx):
    cid = lax.axis_index("c")
    sid = lax.axis_index("s")
    wid = sid * _SC_NC + cid
    hd = Dout // 2
    b = wid // 2
    ch0 = (wid % 2) * hd
    pltpu.sync_copy(mtab_hbm.at[b], tab)
    pltpu.sync_copy(ring_hbm.at[b], rv)
    nch = Nx // _SC_CH

    def gather_chunk(g, obuf):
        @plsc.parallel_loop(0, _SC_CH // _SC_L, unroll=4)
        def _(v):
            rvec = rv[pl.ds(g * _SC_CH + v * _SC_L, _SC_L)]
            rbase = rvec * jnp.full((_SC_L,), Dout, jnp.int32)
            for o in range(hd):
                idx = rbase + jnp.full((_SC_L,), ch0 + o, jnp.int32)
                vals = plsc.load_gather(tab, [idx])
                obuf[o, pl.ds(v * _SC_L, _SC_L)] = vals

    def out_slice(g):
        return out_hbm.at[b, pl.ds(ch0, hd), pl.ds(g * _SC_CH, _SC_CH)]

    def pair_body(gp):
        for half, obuf, sem in ((0, obuf0, sem0), (1, obuf1, sem1)):
            g = gp * 2 + half

            @pl.when(gp > 0)
            def _():
                pltpu.make_async_copy(obuf, out_slice(g), sem).wait()

            gather_chunk(g, obuf)
            pltpu.async_copy(obuf, out_slice(g), sem)

    lax.fori_loop(0, nch // 2, lambda gp, _: (pair_body(gp), 0)[1], 0)
    pltpu.make_async_copy(obuf0, out_slice(nch - 2), sem0).wait()
    pltpu.make_async_copy(obuf1, out_slice(nch - 1), sem1).wait()


def kernel(x, ring, W, b, gamma, beta, mean, var):
    Bx, Din, Nx = x.shape
    R = W.shape[0]
    Dout = W.shape[1]
    nb1 = min(_NB1, Nx)
    nb2 = min(_NB2, Nx)
    nj1 = Nx // nb1
    nj2 = Nx // nb2

    scale = gamma / jnp.sqrt(var + _EPS)
    A = scale[:, :, None] * W
    c = scale * (b - mean) + beta

    Acat = A.reshape(R * Dout, Din)
    ekatze = jnp.zeros((R * Dout, _KAUG - Din), jnp.float32)
    ekatze = ekatze.at[:Dout, 0].set(1.0).at[Dout:, 1].set(1.0)
    Aaug = jnp.concatenate([Acat, ekatze], axis=1).astype(jnp.bfloat16)

    ring = ring.astype(jnp.int32)
    ring3a = ring.reshape(Bx * nj1, 1, nb1)

    m_raw = pl.pallas_call(
        functools.partial(_stage1_body, nb=nb1, nj=nj1),
        grid=(Bx, nj1),
        in_specs=[
            pl.BlockSpec((1, Din, nb1), lambda bi, j: (bi, 0, j)),
            pl.BlockSpec((1, 1, nb1), lambda bi, j, nj=nj1: (bi * nj + j, 0, 0)),
            pl.BlockSpec((R * Dout, _KAUG), lambda bi, j: (0, 0)),
        ],
        out_specs=pl.BlockSpec((1, 1, R * Dout), lambda bi, j: (bi, 0, 0)),
        out_shape=jax.ShapeDtypeStruct((Bx, 1, R * Dout), jnp.float32),
        scratch_shapes=[pltpu.VMEM((R * Dout, 128), jnp.float32)],
        compiler_params=pltpu.CompilerParams(
            dimension_semantics=("arbitrary", "arbitrary"),
        ),
    )(x, ring3a, Aaug)

    Mtab = m_raw.reshape(Bx, R * Dout) + c.reshape(1, R * Dout)

    sc2 = functools.partial(
        pl.kernel,
        mesh=plsc.VectorSubcoreMesh(core_axis_name="c", subcore_axis_name="s"),
        out_type=jax.ShapeDtypeStruct((Bx, Dout, Nx), jnp.float32),
        scratch_types=[
            pltpu.VMEM((Nx,), jnp.int32),
            pltpu.VMEM((Dout // 2, _SC_CH), jnp.float32),
            pltpu.VMEM((Dout // 2, _SC_CH), jnp.float32),
            pltpu.VMEM((R * Dout,), jnp.float32),
            pltpu.SemaphoreType.DMA,
            pltpu.SemaphoreType.DMA,
        ],
        compiler_params=pltpu.CompilerParams(needs_layout_passes=False),
    )(functools.partial(_sc_stage2_body, Bx=Bx, Dout=Dout, Nx=Nx))
    return sc2(Mtab, ring)


# speedup vs baseline: 1.0040x; 1.0040x over previous
import functools
import jax
import jax.numpy as jnp
from jax import lax
from jax.experimental import pallas as pl
from jax.experimental.pallas import tpu as pltpu
from jax.experimental.pallas import tpu_sc as plsc

_EPS = 1e-5
_NB1 = 16384
_NB2 = 4096
_NEG = -1.0e9
_KAUG = 72


def _stage1_body(x_ref, ring_ref, aaug_ref, out_ref, acc_ref, *, nb, nj):
    j = pl.program_id(1)

    xb = x_ref[0].astype(jnp.bfloat16)
    r = ring_ref[0, 0]
    madd0 = jnp.where(r == 0, 0.0, _NEG)
    madd1 = jnp.where(r == 0, _NEG, 0.0)
    srow = jax.lax.broadcasted_iota(jnp.int32, (_KAUG - 64, nb), 0)
    mrows = jnp.where(srow == 0, madd0[None, :],
                      jnp.where(srow == 1, madd1[None, :], 0.0))
    xaug = jnp.concatenate([xb, mrows.astype(jnp.bfloat16)], axis=0)
    h = jnp.dot(aaug_ref[...], xaug, preferred_element_type=jnp.float32)
    parts = [h[:, k * 128:(k + 1) * 128] for k in range(nb // 128)]
    while len(parts) > 1:
        nxt = [jnp.maximum(parts[i], parts[i + 1]) for i in range(0, len(parts) - 1, 2)]
        if len(parts) % 2:
            nxt.append(parts[-1])
        parts = nxt
    p = parts[0]

    @pl.when(j == 0)
    def _():
        acc_ref[...] = p

    @pl.when(j > 0)
    def _():
        acc_ref[...] = jnp.maximum(acc_ref[...], p)

    @pl.when(j == nj - 1)
    def _():
        out_ref[0, 0] = jnp.max(acc_ref[...], axis=1)


_SC_NC = 2
_SC_NS = 16
_SC_L = 16
_SC_CH = 512


def _sc_stage2_body(mtab_hbm, ring_hbm, out_hbm, rv, obuf0, obuf1, tab,
                    sem0, sem1, *, Bx, Dout, Nx):
    cid = lax.axis_index("c")
    sid = lax.axis_index("s")
    wid = sid * _SC_NC + cid
    hd = Dout // 2
    b = wid // 2
    ch0 = (wid % 2) * hd
    pltpu.sync_copy(mtab_hbm.at[b], tab)
    pltpu.sync_copy(ring_hbm.at[b], rv)
    nch = Nx // _SC_CH
    chvec = jnp.zeros((_SC_L,), jnp.int32) + ch0

    def gather_chunk(g, obuf):
        @plsc.parallel_loop(0, _SC_CH // _SC_L, unroll=4)
        def _(v):
            rvec = rv[pl.ds(g * _SC_CH + v * _SC_L, _SC_L)]
            rbase = rvec * jnp.full((_SC_L,), Dout, jnp.int32) + chvec
            for o in range(hd):
                idx = rbase + jnp.full((_SC_L,), o, jnp.int32)
                vals = plsc.load_gather(tab, [idx])
                obuf[o, pl.ds(v * _SC_L, _SC_L)] = vals

    def out_slice(g):
        return out_hbm.at[b, pl.ds(ch0, hd), pl.ds(g * _SC_CH, _SC_CH)]

    def pair_body(gp):
        for half, obuf, sem in ((0, obuf0, sem0), (1, obuf1, sem1)):
            g = gp * 2 + half

            @pl.when(gp > 0)
            def _():
                pltpu.make_async_copy(obuf, out_slice(g), sem).wait()

            gather_chunk(g, obuf)
            pltpu.async_copy(obuf, out_slice(g), sem)

    lax.fori_loop(0, nch // 2, lambda gp, _: (pair_body(gp), 0)[1], 0)
    pltpu.make_async_copy(obuf0, out_slice(nch - 2), sem0).wait()
    pltpu.make_async_copy(obuf1, out_slice(nch - 1), sem1).wait()


def kernel(x, ring, W, b, gamma, beta, mean, var):
    Bx, Din, Nx = x.shape
    R = W.shape[0]
    Dout = W.shape[1]
    nb1 = min(_NB1, Nx)
    nb2 = min(_NB2, Nx)
    nj1 = Nx // nb1
    nj2 = Nx // nb2

    scale = gamma / jnp.sqrt(var + _EPS)
    A = scale[:, :, None] * W
    c = scale * (b - mean) + beta

    Acat = A.reshape(R * Dout, Din)
    ekatze = jnp.zeros((R * Dout, _KAUG - Din), jnp.float32)
    ekatze = ekatze.at[:Dout, 0].set(1.0).at[Dout:, 1].set(1.0)
    Aaug = jnp.concatenate([Acat, ekatze], axis=1).astype(jnp.bfloat16)

    ring = ring.astype(jnp.int32)
    ring3a = ring.reshape(Bx * nj1, 1, nb1)

    m_raw = pl.pallas_call(
        functools.partial(_stage1_body, nb=nb1, nj=nj1),
        grid=(Bx, nj1),
        in_specs=[
            pl.BlockSpec((1, Din, nb1), lambda bi, j: (bi, 0, j)),
            pl.BlockSpec((1, 1, nb1), lambda bi, j, nj=nj1: (bi * nj + j, 0, 0)),
            pl.BlockSpec((R * Dout, _KAUG), lambda bi, j: (0, 0)),
        ],
        out_specs=pl.BlockSpec((1, 1, R * Dout), lambda bi, j: (bi, 0, 0)),
        out_shape=jax.ShapeDtypeStruct((Bx, 1, R * Dout), jnp.float32),
        scratch_shapes=[pltpu.VMEM((R * Dout, 128), jnp.float32)],
        compiler_params=pltpu.CompilerParams(
            dimension_semantics=("arbitrary", "arbitrary"),
        ),
    )(x, ring3a, Aaug)

    Mtab = m_raw.reshape(Bx, R * Dout) + c.reshape(1, R * Dout)

    sc2 = functools.partial(
        pl.kernel,
        mesh=plsc.VectorSubcoreMesh(core_axis_name="c", subcore_axis_name="s"),
        out_type=jax.ShapeDtypeStruct((Bx, Dout, Nx), jnp.float32),
        scratch_types=[
            pltpu.VMEM((Nx,), jnp.int32),
            pltpu.VMEM((Dout // 2, _SC_CH), jnp.float32),
            pltpu.VMEM((Dout // 2, _SC_CH), jnp.float32),
            pltpu.VMEM((R * Dout,), jnp.float32),
            pltpu.SemaphoreType.DMA,
            pltpu.SemaphoreType.DMA,
        ],
        compiler_params=pltpu.CompilerParams(needs_layout_passes=False),
    )(functools.partial(_sc_stage2_body, Bx=Bx, Dout=Dout, Nx=Nx))
    return sc2(Mtab, ring)


# speedup vs baseline: 4.1335x; 4.1169x over previous
import functools
import jax
import jax.numpy as jnp
from jax import lax
from jax.experimental import pallas as pl
from jax.experimental.pallas import tpu as pltpu
from jax.experimental.pallas import tpu_sc as plsc

_EPS = 1e-5
_NB1 = 16384
_NB2 = 4096
_NEG = -1.0e9
_KAUG = 72


def _stage1_body(x_ref, ring_ref, aaug_ref, caug_ref, out_ref, acc_ref, *, nb, nj):
    j = pl.program_id(1)

    xb = x_ref[0].astype(jnp.bfloat16)
    r = ring_ref[0, 0]
    madd0 = jnp.where(r == 0, 0.0, _NEG)
    madd1 = jnp.where(r == 0, _NEG, 0.0)
    srow = jax.lax.broadcasted_iota(jnp.int32, (_KAUG - 64, nb), 0)
    mrows = jnp.where(srow == 0, madd0[None, :],
                      jnp.where(srow == 1, madd1[None, :], 0.0))
    xaug = jnp.concatenate([xb, mrows.astype(jnp.bfloat16)], axis=0)
    h = jnp.dot(aaug_ref[...], xaug, preferred_element_type=jnp.float32)
    parts = [h[:, k * 128:(k + 1) * 128] for k in range(nb // 128)]
    while len(parts) > 1:
        nxt = [jnp.maximum(parts[i], parts[i + 1]) for i in range(0, len(parts) - 1, 2)]
        if len(parts) % 2:
            nxt.append(parts[-1])
        parts = nxt
    p = parts[0]

    @pl.when(j == 0)
    def _():
        acc_ref[...] = p

    @pl.when(j > 0)
    def _():
        acc_ref[...] = jnp.maximum(acc_ref[...], p)

    @pl.when(j == nj - 1)
    def _():
        out_ref[0, 0] = jnp.max(acc_ref[...], axis=1) + caug_ref[0]


_SC_NC = 2
_SC_NS = 16
_SC_L = 16
_SC_CH = 512


def _sc_stage2_body(mtab_hbm, ring_hbm, out_hbm, rv, obuf0, obuf1, tab,
                    sem0, sem1, *, Bx, Dout, Nx):
    cid = lax.axis_index("c")
    sid = lax.axis_index("s")
    wid = sid * _SC_NC + cid
    hd = Dout // 2
    b = wid // 2
    ch0 = (wid % 2) * hd
    pltpu.sync_copy(mtab_hbm.at[b, pl.ds(2 * ch0, 2 * hd)], tab)
    pltpu.sync_copy(ring_hbm.at[b], rv)
    nch = Nx // _SC_CH

    def gather_chunk(g, obuf):
        @plsc.parallel_loop(0, _SC_CH // _SC_L, unroll=4)
        def _(v):
            rvec = rv[pl.ds(g * _SC_CH + v * _SC_L, _SC_L)]
            for o in range(hd):
                idx = rvec + jnp.full((_SC_L,), 2 * o, jnp.int32)
                vals = plsc.load_gather(tab, [idx])
                obuf[o, pl.ds(v * _SC_L, _SC_L)] = vals

    def out_slice(g):
        return out_hbm.at[b, pl.ds(ch0, hd), pl.ds(g * _SC_CH, _SC_CH)]

    def pair_body(gp):
        for half, obuf, sem in ((0, obuf0, sem0), (1, obuf1, sem1)):
            g = gp * 2 + half

            @pl.when(gp > 0)
            def _():
                pltpu.make_async_copy(obuf, out_slice(g), sem).wait()

            gather_chunk(g, obuf)
            pltpu.async_copy(obuf, out_slice(g), sem)

    lax.fori_loop(0, nch // 2, lambda gp, _: (pair_body(gp), 0)[1], 0)
    pltpu.make_async_copy(obuf0, out_slice(nch - 2), sem0).wait()
    pltpu.make_async_copy(obuf1, out_slice(nch - 1), sem1).wait()


def kernel(x, ring, W, b, gamma, beta, mean, var):
    Bx, Din, Nx = x.shape
    R = W.shape[0]
    Dout = W.shape[1]
    nb1 = min(_NB1, Nx)
    nb2 = min(_NB2, Nx)
    nj1 = Nx // nb1
    nj2 = Nx // nb2

    scale = gamma / jnp.sqrt(var + _EPS)
    A = scale[:, :, None] * W
    c = scale * (b - mean) + beta

    Acat = A.reshape(R * Dout, Din)
    ekatze = jnp.zeros((R * Dout, _KAUG - Din), jnp.float32)
    ekatze = ekatze.at[:Dout, 0].set(1.0).at[Dout:, 1].set(1.0)
    Aaug = jnp.concatenate([Acat, ekatze], axis=1).astype(jnp.bfloat16)
    perm = (jnp.arange(R * Dout) % R) * Dout + jnp.arange(R * Dout) // R
    Aaug = Aaug[perm]
    caug = jnp.transpose(c, (1, 0)).reshape(1, R * Dout)

    ring = ring.astype(jnp.int32)
    ring3a = ring.reshape(Bx * nj1, 1, nb1)

    m_raw = pl.pallas_call(
        functools.partial(_stage1_body, nb=nb1, nj=nj1),
        grid=(Bx, nj1),
        in_specs=[
            pl.BlockSpec((1, Din, nb1), lambda bi, j: (bi, 0, j)),
            pl.BlockSpec((1, 1, nb1), lambda bi, j, nj=nj1: (bi * nj + j, 0, 0)),
            pl.BlockSpec((R * Dout, _KAUG), lambda bi, j: (0, 0)),
            pl.BlockSpec((1, R * Dout), lambda bi, j: (0, 0)),
        ],
        out_specs=pl.BlockSpec((1, 1, R * Dout), lambda bi, j: (bi, 0, 0)),
        out_shape=jax.ShapeDtypeStruct((Bx, 1, R * Dout), jnp.float32),
        scratch_shapes=[pltpu.VMEM((R * Dout, 128), jnp.float32)],
        compiler_params=pltpu.CompilerParams(
            dimension_semantics=("arbitrary", "arbitrary"),
        ),
    )(x, ring3a, Aaug, caug)

    Mtab = m_raw.reshape(Bx, R * Dout)

    sc2 = functools.partial(
        pl.kernel,
        mesh=plsc.VectorSubcoreMesh(core_axis_name="c", subcore_axis_name="s"),
        out_type=jax.ShapeDtypeStruct((Bx, Dout, Nx), jnp.float32),
        scratch_types=[
            pltpu.VMEM((Nx,), jnp.int32),
            pltpu.VMEM((Dout // 2, _SC_CH), jnp.float32),
            pltpu.VMEM((Dout // 2, _SC_CH), jnp.float32),
            pltpu.VMEM((R * Dout // 2,), jnp.float32),
            pltpu.SemaphoreType.DMA,
            pltpu.SemaphoreType.DMA,
        ],
        compiler_params=pltpu.CompilerParams(needs_layout_passes=False),
    )(functools.partial(_sc_stage2_body, Bx=Bx, Dout=Dout, Nx=Nx))
    return sc2(Mtab, ring)
